# baseline (speedup 1.0000x reference)
import jax
import jax.numpy as jnp
from jax.experimental import pallas as pl

_S = 1.0 / (1.0 + 1e-3) ** 0.5


def _proj_body(e_ref, wf_ref, wg_ref, pf_ref, pg_ref):
    e = e_ref[...]
    pf_ref[...] = e @ wf_ref[...]
    pg_ref[...] = e @ wg_ref[...]


def _msg_body(sbfm_ref, sbfk_ref, wam_ref, wak_ref,
              fm1_ref, fm2_ref, fk1_ref, fk2_ref,
              gm1_ref, gm2_ref, gk1_ref, gk2_ref,
              msgm_ref, msgk_ref):
    am = sbfm_ref[...] @ wam_ref[...]
    ak = sbfk_ref[...] @ wak_ref[...]
    h1m = fm1_ref[...] + fm2_ref[...]
    h1k = fk1_ref[...] + fk2_ref[...]
    h0m = gm1_ref[:, 0:1] + gm2_ref[:, 1:2]
    h0k = gk1_ref[:, 2:3] + gk2_ref[:, 3:4]
    msgm_ref[...] = am * h0m * h1m
    msgk_ref[...] = ak * h0k * h1k


def _final_body(e_ref, sm_ref, sk_ref, wpm_ref, wpk_ref,
                w0a_ref, b0a_ref, w0b_ref, b0b_ref,
                w1a_ref, b1a_ref, w1b_ref, b1b_ref, out_ref):
    x = e_ref[...] + sm_ref[...] @ wpm_ref[...] + sk_ref[...] @ wpk_ref[...]
    x = x + ((x @ w0a_ref[...] + b0a_ref[...]) @ w0b_ref[...] + b0b_ref[...])
    x = x + ((x @ w1a_ref[...] + b1a_ref[...]) @ w1b_ref[...] + b1b_ref[...])
    out_ref[...] = x


def kernel(bond_embedding, sbf_mij, sbf_kji, W_im1, W_im2, W_kj1, W_kj2,
           Wa_mij1, Wa_mij2, Wa_kji1, Wa_kji2, W_pre,
           Wr0a, br0a, Wr0b, br0b, Wr1a, br1a, Wr1b, br1b,
           bond_mi_id_for_angle_mij_list, bond_ij_id_for_angle_mij_list,
           bond_kj_id_for_angle_kji_list, bond_ij_id_for_angle_kji_list):
    e = bond_embedding
    nb, h = e.shape
    na = sbf_mij.shape[0]
    f32 = jnp.float32

    s2 = jnp.float32(_S * _S)
    Wim = (W_im1 @ W_im2) * s2
    Wkj = (W_kj1 @ W_kj2) * s2
    A_mi, A_ij = Wim[:h], Wim[h:]
    A_kj, A_ij2 = Wkj[:h], Wkj[h:]
    WF = jnp.concatenate([A_mi[:, 1:], A_ij[:, 1:], A_kj[:, 1:], A_ij2[:, 1:]],
                         axis=1)
    gates = jnp.stack([A_mi[:, 0], A_ij[:, 0], A_kj[:, 0], A_ij2[:, 0]],
                      axis=1)
    WG = jnp.concatenate([gates, jnp.zeros((h, 12), f32)], axis=1)
    Wam = Wa_mij1 @ Wa_mij2
    Wak = Wa_kji1 @ Wa_kji2
    Wp = W_pre * jnp.float32(_S)
    Wp_m, Wp_k = Wp[:h], Wp[h:]

    BLK = 2000
    nb_steps = nb // BLK
    pf, pg = pl.pallas_call(
        _proj_body,
        grid=(nb_steps,),
        in_specs=[
            pl.BlockSpec((BLK, h), lambda i: (i, 0)),
            pl.BlockSpec((h, 4 * h), lambda i: (0, 0)),
            pl.BlockSpec((h, 16), lambda i: (0, 0)),
        ],
        out_specs=[
            pl.BlockSpec((BLK, 4 * h), lambda i: (i, 0)),
            pl.BlockSpec((BLK, 16), lambda i: (i, 0)),
        ],
        out_shape=[
            jax.ShapeDtypeStruct((nb, 4 * h), f32),
            jax.ShapeDtypeStruct((nb, 16), f32),
        ],
    )(e, WF, WG)

    Fa = pf[:, 0 * h:1 * h]
    Fb = pf[:, 1 * h:2 * h]
    Fc = pf[:, 2 * h:3 * h]
    Fd = pf[:, 3 * h:4 * h]

    mi = bond_mi_id_for_angle_mij_list
    ijm = bond_ij_id_for_angle_mij_list
    kj = bond_kj_id_for_angle_kji_list
    ijk = bond_ij_id_for_angle_kji_list

    fm1 = jnp.take(Fa, mi, axis=0)
    fm2 = jnp.take(Fb, ijm, axis=0)
    fk1 = jnp.take(Fc, kj, axis=0)
    fk2 = jnp.take(Fd, ijk, axis=0)
    gm1 = jnp.take(pg, mi, axis=0)
    gm2 = jnp.take(pg, ijm, axis=0)
    gk1 = jnp.take(pg, kj, axis=0)
    gk2 = jnp.take(pg, ijk, axis=0)

    BLA = 2000
    na_steps = na // BLA
    row = lambda i: (i, 0)
    full = lambda shape: pl.BlockSpec(shape, lambda i: (0, 0))
    msgm, msgk = pl.pallas_call(
        _msg_body,
        grid=(na_steps,),
        in_specs=[
            pl.BlockSpec((BLA, 16), row), pl.BlockSpec((BLA, 16), row),
            full((16, h)), full((16, h)),
            pl.BlockSpec((BLA, h), row), pl.BlockSpec((BLA, h), row),
            pl.BlockSpec((BLA, h), row), pl.BlockSpec((BLA, h), row),
            pl.BlockSpec((BLA, 16), row), pl.BlockSpec((BLA, 16), row),
            pl.BlockSpec((BLA, 16), row), pl.BlockSpec((BLA, 16), row),
        ],
        out_specs=[
            pl.BlockSpec((BLA, h), row), pl.BlockSpec((BLA, h), row),
        ],
        out_shape=[
            jax.ShapeDtypeStruct((na, h), f32),
            jax.ShapeDtypeStruct((na, h), f32),
        ],
    )(sbf_mij, sbf_kji, Wam, Wak, fm1, fm2, fk1, fk2, gm1, gm2, gk1, gk2)

    sum_m = jax.ops.segment_sum(msgm, ijm, num_segments=nb)
    sum_k = jax.ops.segment_sum(msgk, ijk, num_segments=nb)

    b2 = lambda: pl.BlockSpec((1, h), lambda i: (0, 0))
    wfull = lambda: pl.BlockSpec((h, h), lambda i: (0, 0))
    out = pl.pallas_call(
        _final_body,
        grid=(nb_steps,),
        in_specs=[
            pl.BlockSpec((BLK, h), row), pl.BlockSpec((BLK, h), row),
            pl.BlockSpec((BLK, h), row),
            wfull(), wfull(),
            wfull(), b2(), wfull(), b2(),
            wfull(), b2(), wfull(), b2(),
        ],
        out_specs=pl.BlockSpec((BLK, h), row),
        out_shape=jax.ShapeDtypeStruct((nb, h), f32),
    )(e, sum_m, sum_k, Wp_m, Wp_k,
      Wr0a, br0a.reshape(1, h), Wr0b, br0b.reshape(1, h),
      Wr1a, br1a.reshape(1, h), Wr1b, br1b.reshape(1, h))
    return out


# speedup vs baseline: 1.1229x; 1.1229x over previous
import jax
import jax.numpy as jnp
from jax import lax
from jax.experimental import pallas as pl
from jax.experimental.pallas import tpu as pltpu
from jax.experimental.pallas import tpu_sc as plsc

_S = 1.0 / (1.0 + 1e-3) ** 0.5
_NC = 2
_NS = 16
_NW = _NC * _NS
_C = 80


def _proj_body(e_ref, wa_ref, wb_ref, wc_ref, wd_ref, wg_ref,
               ta_ref, tb_ref, tc_ref, td_ref, pg_ref):
    e = e_ref[...]
    ta_ref[...] = e @ wa_ref[...]
    tb_ref[...] = e @ wb_ref[...]
    tc_ref[...] = e @ wc_ref[...]
    td_ref[...] = e @ wd_ref[...]
    pg_ref[...] = e @ wg_ref[...]


def _gather_body(ta, tb, tc, td, mi, ijm, kj, ijk, hm, hk,
                 idx1, idx2, r1, r2, sem):
    wid = lax.axis_index("s") * _NC + lax.axis_index("c")
    per_w = mi.shape[0] // _NW
    nchunk = per_w // _C
    base0 = wid * per_w

    for t1, t2, i1, i2, out in ((ta, tb, mi, ijm, hm),
                                (tc, td, kj, ijk, hk)):
        def chunk_body(c, _, t1=t1, t2=t2, i1=i1, i2=i2, out=out):
            base = base0 + c * _C
            pltpu.sync_copy(i1.at[pl.ds(base, _C)], idx1)
            pltpu.sync_copy(i2.at[pl.ds(base, _C)], idx2)
            cp1 = pltpu.async_copy(t1.at[idx1], r1, sem)
            cp2 = pltpu.async_copy(t2.at[idx2], r2, sem)
            cp1.wait()
            cp2.wait()

            def row_add(i, __):
                for j in range(8):
                    sl = pl.ds(j * 16, 16)
                    r1[i, sl] = r1[i, sl] + r2[i, sl]
                return 0

            lax.fori_loop(0, _C, row_add, 0)
            pltpu.sync_copy(r1, out.at[pl.ds(base, _C)])
            return 0

        lax.fori_loop(0, nchunk, chunk_body, 0)


def _msg_body(sbfm_ref, sbfk_ref, wam_ref, wak_ref, h1m_ref, h1k_ref,
              gm1_ref, gm2_ref, gk1_ref, gk2_ref, msgm_ref, msgk_ref):
    am = sbfm_ref[...] @ wam_ref[...]
    ak = sbfk_ref[...] @ wak_ref[...]
    h0m = gm1_ref[:, 0:1] + gm2_ref[:, 1:2]
    h0k = gk1_ref[:, 2:3] + gk2_ref[:, 3:4]
    msgm_ref[...] = am * h0m * h1m_ref[...]
    msgk_ref[...] = ak * h0k * h1k_ref[...]


def _final_body(e_ref, sm_ref, sk_ref, wpm_ref, wpk_ref,
                w0a_ref, b0a_ref, w0b_ref, b0b_ref,
                w1a_ref, b1a_ref, w1b_ref, b1b_ref, out_ref):
    x = e_ref[...] + sm_ref[...] @ wpm_ref[...] + sk_ref[...] @ wpk_ref[...]
    x = x + ((x @ w0a_ref[...] + b0a_ref[...]) @ w0b_ref[...] + b0b_ref[...])
    x = x + ((x @ w1a_ref[...] + b1a_ref[...]) @ w1b_ref[...] + b1b_ref[...])
    out_ref[...] = x


def kernel(bond_embedding, sbf_mij, sbf_kji, W_im1, W_im2, W_kj1, W_kj2,
           Wa_mij1, Wa_mij2, Wa_kji1, Wa_kji2, W_pre,
           Wr0a, br0a, Wr0b, br0b, Wr1a, br1a, Wr1b, br1b,
           bond_mi_id_for_angle_mij_list, bond_ij_id_for_angle_mij_list,
           bond_kj_id_for_angle_kji_list, bond_ij_id_for_angle_kji_list):
    e = bond_embedding
    nb, h = e.shape
    na = sbf_mij.shape[0]
    f32 = jnp.float32

    s2 = jnp.float32(_S * _S)
    Wim = (W_im1 @ W_im2) * s2
    Wkj = (W_kj1 @ W_kj2) * s2
    A_mi, A_ij = Wim[:h], Wim[h:]
    A_kj, A_ij2 = Wkj[:h], Wkj[h:]
    gates = jnp.stack([A_mi[:, 0], A_ij[:, 0], A_kj[:, 0], A_ij2[:, 0]],
                      axis=1)
    WG = jnp.concatenate([gates, jnp.zeros((h, 12), f32)], axis=1)
    Wam = Wa_mij1 @ Wa_mij2
    Wak = Wa_kji1 @ Wa_kji2
    Wp = W_pre * jnp.float32(_S)
    Wp_m, Wp_k = Wp[:h], Wp[h:]

    BLK = 2000
    nb_steps = nb // BLK
    row = lambda i: (i, 0)
    wspec = pl.BlockSpec((h, h), lambda i: (0, 0))
    tspec = pl.BlockSpec((BLK, h), row)
    tshape = jax.ShapeDtypeStruct((nb, h), f32)
    ta, tb, tc, td, pg = pl.pallas_call(
        _proj_body,
        grid=(nb_steps,),
        in_specs=[pl.BlockSpec((BLK, h), row), wspec, wspec, wspec, wspec,
                  pl.BlockSpec((h, 16), lambda i: (0, 0))],
        out_specs=[tspec, tspec, tspec, tspec,
                   pl.BlockSpec((BLK, 16), row)],
        out_shape=[tshape, tshape, tshape, tshape,
                   jax.ShapeDtypeStruct((nb, 16), f32)],
    )(e, A_mi[:, 1:], A_ij[:, 1:], A_kj[:, 1:], A_ij2[:, 1:], WG)

    mi = bond_mi_id_for_angle_mij_list
    ijm = bond_ij_id_for_angle_mij_list
    kj = bond_kj_id_for_angle_kji_list
    ijk = bond_ij_id_for_angle_kji_list

    mesh = plsc.VectorSubcoreMesh(core_axis_name="c", subcore_axis_name="s")
    hshape = jax.ShapeDtypeStruct((na, h), f32)
    h1m, h1k = pl.kernel(
        _gather_body,
        out_type=[hshape, hshape],
        mesh=mesh,
        scratch_types=[
            pltpu.VMEM((_C,), jnp.int32),
            pltpu.VMEM((_C,), jnp.int32),
            pltpu.VMEM((_C, h), f32),
            pltpu.VMEM((_C, h), f32),
            pltpu.SemaphoreType.DMA,
        ],
    )(ta, tb, tc, td, mi, ijm, kj, ijk)

    gm1 = jnp.take(pg, mi, axis=0)
    gm2 = jnp.take(pg, ijm, axis=0)
    gk1 = jnp.take(pg, kj, axis=0)
    gk2 = jnp.take(pg, ijk, axis=0)

    BLA = 2000
    na_steps = na // BLA
    msgm, msgk = pl.pallas_call(
        _msg_body,
        grid=(na_steps,),
        in_specs=[
            pl.BlockSpec((BLA, 16), row), pl.BlockSpec((BLA, 16), row),
            pl.BlockSpec((16, h), lambda i: (0, 0)),
            pl.BlockSpec((16, h), lambda i: (0, 0)),
            pl.BlockSpec((BLA, h), row), pl.BlockSpec((BLA, h), row),
            pl.BlockSpec((BLA, 16), row), pl.BlockSpec((BLA, 16), row),
            pl.BlockSpec((BLA, 16), row), pl.BlockSpec((BLA, 16), row),
        ],
        out_specs=[pl.BlockSpec((BLA, h), row), pl.BlockSpec((BLA, h), row)],
        out_shape=[
            jax.ShapeDtypeStruct((na, h), f32),
            jax.ShapeDtypeStruct((na, h), f32),
        ],
    )(sbf_mij, sbf_kji, Wam, Wak, h1m, h1k, gm1, gm2, gk1, gk2)

    sum_m = jax.ops.segment_sum(msgm, ijm, num_segments=nb)
    sum_k = jax.ops.segment_sum(msgk, ijk, num_segments=nb)

    b2 = lambda: pl.BlockSpec((1, h), lambda i: (0, 0))
    wfull = lambda: pl.BlockSpec((h, h), lambda i: (0, 0))
    out = pl.pallas_call(
        _final_body,
        grid=(nb_steps,),
        in_specs=[
            pl.BlockSpec((BLK, h), row), pl.BlockSpec((BLK, h), row),
            pl.BlockSpec((BLK, h), row),
            wfull(), wfull(),
            wfull(), b2(), wfull(), b2(),
            wfull(), b2(), wfull(), b2(),
        ],
        out_specs=pl.BlockSpec((BLK, h), row),
        out_shape=jax.ShapeDtypeStruct((nb, h), f32),
    )(e, sum_m, sum_k, Wp_m, Wp_k,
      Wr0a, br0a.reshape(1, h), Wr0b, br0b.reshape(1, h),
      Wr1a, br1a.reshape(1, h), Wr1b, br1b.reshape(1, h))
    return out
